# initial kernel scaffold (unmeasured)
import jax
import jax.numpy as jnp
from jax import lax
from jax.experimental import pallas as pl
from jax.experimental.pallas import tpu as pltpu

N_DEV = 8
HOPS = N_DEV - 1
T = 4


def kernel(x, w_mat):
    m_global, k_shard = x.shape
    _, n = w_mat.shape
    m_chunk = m_global // N_DEV
    tw = n // T

    def body(x_hbm, w_ref, out_ref, comm, xbuf,
             send_sems, recv_sems, x_sems, out_sems, credit_sem):
        my = lax.axis_index("i")
        left = lax.rem(my + N_DEV - 1, N_DEV)
        right = lax.rem(my + 1, N_DEV)

        barrier = pltpu.get_barrier_semaphore()
        for nbr in (left, right):
            pl.semaphore_signal(barrier, inc=1, device_id=(nbr,),
                                device_id_type=pl.DeviceIdType.MESH)
        pl.semaphore_wait(barrier, 2)

        def chunk_id(i):
            if i < HOPS:
                return lax.rem(my + (HOPS - i), N_DEV)
            return my

        def x_copy(i):
            c = chunk_id(i)
            return pltpu.make_async_copy(
                x_hbm.at[pl.ds(c * m_chunk, m_chunk), :],
                xbuf.at[i % 2],
                x_sems.at[i],
            )

        def rdma(t, s):
            return pltpu.make_async_remote_copy(
                src_ref=comm.at[t, s % 2],
                dst_ref=comm.at[t, (s + 1) % 2],
                send_sem=send_sems.at[t, s],
                recv_sem=recv_sems.at[t, s],
                device_id=(right,),
                device_id_type=pl.DeviceIdType.MESH,
            )

        x_copy(0).start()

        for s in range(HOPS):
            for t in range(T):
                g = s * T + t
                if t == 0:
                    x_copy(s + 1).start()
                    x_copy(s).wait()
                if s >= 1:
                    rdma(t, s - 1).wait_recv()
                p = jnp.dot(xbuf[s % 2], w_ref[:, t * tw:(t + 1) * tw],
                            preferred_element_type=jnp.float32)
                if s == 0:
                    comm[t, 0] = p
                else:
                    comm[t, s % 2] += p
                if s >= 1:
                    pl.semaphore_wait(credit_sem, 1)
                rdma(t, s).start()
                if g >= 1:
                    ps, pt = (s, t - 1) if t >= 1 else (s - 1, T - 1)
                    rdma(pt, ps).wait_send()
                    if ps <= HOPS - 2:
                        pl.semaphore_signal(credit_sem, inc=1,
                                            device_id=(left,),
                                            device_id_type=pl.DeviceIdType.MESH)

        rdma(T - 1, HOPS - 1).wait_send()
        x_copy(HOPS).wait()
        for t in range(T):
            rdma(t, HOPS - 1).wait_recv()
            p = jnp.dot(xbuf[HOPS % 2], w_ref[:, t * tw:(t + 1) * tw],
                        preferred_element_type=jnp.float32)
            comm[t, 0] = jnp.maximum(comm[t, 1] + p, 0.0)
            pltpu.make_async_copy(
                comm.at[t, 0], out_ref.at[:, pl.ds(t * tw, tw)], out_sems.at[t]
            ).start()
        for t in range(T):
            pltpu.make_async_copy(
                comm.at[t, 0], out_ref.at[:, pl.ds(t * tw, tw)], out_sems.at[t]
            ).wait()

    return pl.pallas_call(
        body,
        out_shape=jax.ShapeDtypeStruct((m_chunk, n), jnp.float32),
        in_specs=[
            pl.BlockSpec(memory_space=pl.ANY),
            pl.BlockSpec(memory_space=pltpu.VMEM),
        ],
        out_specs=pl.BlockSpec(memory_space=pl.ANY),
        scratch_shapes=[
            pltpu.VMEM((T, 2, m_chunk, tw), jnp.float32),
            pltpu.VMEM((2, m_chunk, k_shard), jnp.float32),
            pltpu.SemaphoreType.DMA((T, HOPS)),
            pltpu.SemaphoreType.DMA((T, HOPS)),
            pltpu.SemaphoreType.DMA((HOPS + 1,)),
            pltpu.SemaphoreType.DMA((T,)),
            pltpu.SemaphoreType.REGULAR,
        ],
        compiler_params=pltpu.CompilerParams(collective_id=0),
    )(x, w_mat)


# baseline (device time: 1295804 ns/iter reference)
import jax
import jax.numpy as jnp
from jax import lax
from jax.experimental import pallas as pl
from jax.experimental.pallas import tpu as pltpu

N_DEV = 8
HOPS = N_DEV - 1
T = 4


def kernel(x, w_mat):
    m_global, k_shard = x.shape
    _, n = w_mat.shape
    m_chunk = m_global // N_DEV
    tw = n // T

    def body(x_hbm, w_ref, out_ref, comm, xbuf,
             send_sems, recv_sems, x_sems, out_sems, credit_sem):
        my = lax.axis_index("i")
        left = lax.rem(my + N_DEV - 1, N_DEV)
        right = lax.rem(my + 1, N_DEV)

        barrier = pltpu.get_barrier_semaphore()
        for nbr in (left, right):
            pl.semaphore_signal(barrier, inc=1, device_id=(nbr,),
                                device_id_type=pl.DeviceIdType.MESH)
        pl.semaphore_wait(barrier, 2)

        def chunk_id(i):
            if i < HOPS:
                return lax.rem(my + (HOPS - i), N_DEV)
            return my

        def x_copy(i):
            c = chunk_id(i)
            return pltpu.make_async_copy(
                x_hbm.at[pl.ds(c * m_chunk, m_chunk), :],
                xbuf.at[i % 2],
                x_sems.at[i],
            )

        def rdma(t, s):
            return pltpu.make_async_remote_copy(
                src_ref=comm.at[t, s % 2],
                dst_ref=comm.at[t, (s + 1) % 2],
                send_sem=send_sems.at[t, s],
                recv_sem=recv_sems.at[t, s],
                device_id=(right,),
                device_id_type=pl.DeviceIdType.MESH,
            )

        x_copy(0).start()

        for s in range(HOPS):
            for t in range(T):
                g = s * T + t
                if t == 0:
                    x_copy(s + 1).start()
                    x_copy(s).wait()
                if s >= 1:
                    rdma(t, s - 1).wait_recv()
                p = jnp.dot(xbuf[s % 2], w_ref[:, t * tw:(t + 1) * tw],
                            preferred_element_type=jnp.float32)
                if s == 0:
                    comm[t, 0] = p
                else:
                    comm[t, s % 2] += p
                if s >= 1:
                    pl.semaphore_wait(credit_sem, 1)
                rdma(t, s).start()
                if g >= 1:
                    ps, pt = (s, t - 1) if t >= 1 else (s - 1, T - 1)
                    rdma(pt, ps).wait_send()
                    if ps <= HOPS - 2:
                        pl.semaphore_signal(credit_sem, inc=1,
                                            device_id=(left,),
                                            device_id_type=pl.DeviceIdType.MESH)

        rdma(T - 1, HOPS - 1).wait_send()
        x_copy(HOPS).wait()
        for t in range(T):
            rdma(t, HOPS - 1).wait_recv()
            p = jnp.dot(xbuf[HOPS % 2], w_ref[:, t * tw:(t + 1) * tw],
                        preferred_element_type=jnp.float32)
            comm[t, 0] = jnp.maximum(comm[t, 1] + p, 0.0)
            pltpu.make_async_copy(
                comm.at[t, 0], out_ref.at[:, pl.ds(t * tw, tw)], out_sems.at[t]
            ).start()
        for t in range(T):
            pltpu.make_async_copy(
                comm.at[t, 0], out_ref.at[:, pl.ds(t * tw, tw)], out_sems.at[t]
            ).wait()

    return pl.pallas_call(
        body,
        out_shape=jax.ShapeDtypeStruct((m_chunk, n), jnp.float32),
        in_specs=[
            pl.BlockSpec(memory_space=pl.ANY),
            pl.BlockSpec(memory_space=pltpu.VMEM),
        ],
        out_specs=pl.BlockSpec(memory_space=pl.ANY),
        scratch_shapes=[
            pltpu.VMEM((T, 2, m_chunk, tw), jnp.float32),
            pltpu.VMEM((2, m_chunk, k_shard), jnp.float32),
            pltpu.SemaphoreType.DMA((T, HOPS)),
            pltpu.SemaphoreType.DMA((T, HOPS)),
            pltpu.SemaphoreType.DMA((HOPS + 1,)),
            pltpu.SemaphoreType.DMA((T,)),
            pltpu.SemaphoreType.REGULAR,
        ],
        compiler_params=pltpu.CompilerParams(
            collective_id=0, vmem_limit_bytes=100 * 1024 * 1024
        ),
    )(x, w_mat)
